# baseline (speedup 1.0000x reference)
import functools

import jax
import jax.numpy as jnp
from jax import lax
from jax.experimental import pallas as pl
from jax.experimental.pallas import tpu as pltpu
from jax.experimental.pallas import tpu_sc as plsc

_CHUNK = 16
_NBUF = 4
_NSLOT = 3


def _embedding_lookup(tokens_flat, table):
    B, = tokens_flat.shape
    V, D = table.shape
    info = plsc.get_sparse_core_info()
    NC, NS = info.num_cores, info.num_subcores
    NW = NC * NS
    b_per_w = B // NW
    n_chunks = b_per_w // _CHUNK
    assert B == NW * b_per_w and b_per_w == n_chunks * _CHUNK
    assert n_chunks % _NBUF == 0 and _NBUF >= _NSLOT

    mesh = plsc.VectorSubcoreMesh(core_axis_name="c", subcore_axis_name="s")

    @functools.partial(
        pl.kernel,
        mesh=mesh,
        out_type=jax.ShapeDtypeStruct((B, D), jnp.float32),
        scratch_types=[
            pltpu.VMEM((b_per_w,), jnp.int32),
        ]
        + [pltpu.VMEM((_CHUNK, D), jnp.float32)] * _NBUF
        + [pltpu.VMEM_SHARED((NS, _NSLOT, _CHUNK, D), jnp.float32)]
        + [pltpu.SemaphoreType.DMA] * (2 * _NBUF + _NSLOT),
    )
    def gather_kernel(idx_hbm, table_hbm, out_hbm, idx_v, *bufs_sems):
        bufs = bufs_sems[:_NBUF]
        shared = bufs_sems[_NBUF]
        gsems = bufs_sems[_NBUF + 1:2 * _NBUF + 1]
        ssems = bufs_sems[2 * _NBUF + 1:3 * _NBUF + 1]
        dsems = bufs_sems[3 * _NBUF + 1:]
        sid = lax.axis_index("s")
        wid = sid * NC + lax.axis_index("c")
        base = wid * b_per_w
        pltpu.sync_copy(idx_hbm.at[pl.ds(base, b_per_w)], idx_v)

        def out_slice(i):
            return out_hbm.at[pl.ds(base + i * _CHUNK, _CHUNK)]

        def slot(s):
            return shared.at[sid, s]

        def start_gather(i, b):
            off = pl.multiple_of(i * _CHUNK, _CHUNK)
            pltpu.async_copy(table_hbm.at[idx_v.at[pl.ds(off, _CHUNK)]],
                             bufs[b], gsems[b])

        for b in range(_NBUF - 1):
            start_gather(b, b)

        def step(i, b, s, first=False, last=False):
            pltpu.make_async_copy(table_hbm.at[idx_v.at[pl.ds(0, _CHUNK)]],
                                  bufs[b], gsems[b]).wait()
            if not last:
                start_gather(i + _NBUF - 1, (b + _NBUF - 1) % _NBUF)
            if not first:
                pltpu.make_async_copy(slot(s), out_slice(0), dsems[s]).wait()
            pltpu.async_copy(bufs[b], slot(s), ssems[b])
            pltpu.make_async_copy(bufs[b], slot(s), ssems[b]).wait()
            pltpu.async_copy(slot(s), out_slice(i), dsems[s])

        for i in range(_NSLOT):
            step(i, i % _NBUF, i % _NSLOT, first=True)

        n_steady = (n_chunks - 2 * _NBUF) // _NBUF

        def body(grp, carry):
            for k in range(_NBUF):
                i = _NSLOT + _NBUF * grp + k
                step(i, (_NSLOT + k) % _NBUF, (_NSLOT + k) % _NSLOT)
            return carry

        lax.fori_loop(0, n_steady, body, 0)

        for i in range(_NSLOT + n_steady * _NBUF, n_chunks):
            step(i, i % _NBUF, i % _NSLOT, last=(i + _NBUF - 1 >= n_chunks))

        for s in range(_NSLOT):
            pltpu.make_async_copy(slot(s), out_slice(0), dsems[s]).wait()

    return gather_kernel(tokens_flat, table)


def kernel(tokens, start_pos, tok_embeddings_weight):
    B, S = tokens.shape
    V, D = tok_embeddings_weight.shape
    out = _embedding_lookup(tokens.reshape(B * S), tok_embeddings_weight)
    return out.reshape(B, S, D)


# speedup vs baseline: 1.0047x; 1.0047x over previous
import functools

import jax
import jax.numpy as jnp
from jax import lax
from jax.experimental import pallas as pl
from jax.experimental.pallas import tpu as pltpu
from jax.experimental.pallas import tpu_sc as plsc

_CHUNK = 16
_NBUF = 4
_NSLOT = 3


def _embedding_lookup(tokens_flat, table):
    B, = tokens_flat.shape
    V, D = table.shape
    info = plsc.get_sparse_core_info()
    NC, NS = info.num_cores, info.num_subcores
    NW = NC * NS
    b_per_w = B // NW
    n_chunks = b_per_w // _CHUNK
    assert B == NW * b_per_w and b_per_w == n_chunks * _CHUNK
    assert n_chunks % _NBUF == 0 and _NBUF >= _NSLOT

    mesh = plsc.VectorSubcoreMesh(core_axis_name="c", subcore_axis_name="s")

    @functools.partial(
        pl.kernel,
        mesh=mesh,
        out_type=jax.ShapeDtypeStruct((B, D), jnp.float32),
        scratch_types=[
            pltpu.VMEM((b_per_w,), jnp.int32),
        ]
        + [pltpu.VMEM((_CHUNK, D), jnp.float32)] * _NBUF
        + [pltpu.VMEM_SHARED((NS, _NSLOT, _CHUNK, D), jnp.float32)]
        + [pltpu.SemaphoreType.DMA] * (2 * _NBUF + _NSLOT),
    )
    def gather_kernel(idx_hbm, table_hbm, out_hbm, idx_v, *bufs_sems):
        bufs = bufs_sems[:_NBUF]
        shared = bufs_sems[_NBUF]
        gsems = bufs_sems[_NBUF + 1:2 * _NBUF + 1]
        ssems = bufs_sems[2 * _NBUF + 1:3 * _NBUF + 1]
        dsems = bufs_sems[3 * _NBUF + 1:]
        sid = lax.axis_index("s")
        wid = sid * NC + lax.axis_index("c")
        base = wid * b_per_w
        pltpu.sync_copy(idx_hbm.at[pl.ds(base, b_per_w)], idx_v)

        def out_slice(i):
            return out_hbm.at[pl.ds(base + i * _CHUNK, _CHUNK)]

        def slot(s):
            return shared.at[sid, s]

        def start_gather(i, b):
            off = pl.multiple_of(i * _CHUNK, _CHUNK)
            pltpu.async_copy(table_hbm.at[idx_v.at[pl.ds(off, _CHUNK)]],
                             bufs[b], gsems[b])

        for b in range(_NBUF - 1):
            start_gather(b, b)

        def step(i, b, s, pb, ps, has_prev, slot_busy, last):
            pltpu.make_async_copy(table_hbm.at[idx_v.at[pl.ds(0, _CHUNK)]],
                                  bufs[b], gsems[b]).wait()
            if has_prev:
                pltpu.make_async_copy(bufs[pb], slot(ps), ssems[pb]).wait()
            if not last:
                start_gather(i + _NBUF - 1, pb)
            if has_prev:
                pltpu.async_copy(slot(ps), out_slice(i - 1), dsems[ps])
            if slot_busy:
                pltpu.make_async_copy(slot(s), out_slice(0), dsems[s]).wait()
            pltpu.async_copy(bufs[b], slot(s), ssems[b])

        def full_step(i, last=False):
            step(i, i % _NBUF, i % _NSLOT, (i - 1) % _NBUF,
                 (i - 1) % _NSLOT, has_prev=(i >= 1),
                 slot_busy=(i >= _NSLOT), last=last)

        period = 12
        lead = _NSLOT
        n_steady = (n_chunks - lead - (_NBUF - 1) - 10) // period

        for i in range(lead):
            full_step(i)

        def body(grp, carry):
            for k in range(period):
                j = lead + k
                step(lead + period * grp + k, j % _NBUF, j % _NSLOT,
                     (j - 1) % _NBUF, (j - 1) % _NSLOT,
                     has_prev=True, slot_busy=True, last=False)
            return carry

        lax.fori_loop(0, n_steady, body, 0)

        for i in range(lead + n_steady * period, n_chunks):
            full_step(i, last=(i + _NBUF - 1 >= n_chunks))

        last_b = (n_chunks - 1) % _NBUF
        last_s = (n_chunks - 1) % _NSLOT
        pltpu.make_async_copy(bufs[last_b], slot(last_s),
                              ssems[last_b]).wait()
        pltpu.async_copy(slot(last_s), out_slice(n_chunks - 1), dsems[last_s])
        for s in range(_NSLOT):
            pltpu.make_async_copy(slot(s), out_slice(0), dsems[s]).wait()

    return gather_kernel(tokens_flat, table)


def kernel(tokens, start_pos, tok_embeddings_weight):
    B, S = tokens.shape
    V, D = tok_embeddings_weight.shape
    out = _embedding_lookup(tokens.reshape(B * S), tok_embeddings_weight)
    return out.reshape(B, S, D)


# speedup vs baseline: 1.0113x; 1.0066x over previous
import functools

import jax
import jax.numpy as jnp
from jax import lax
from jax.experimental import pallas as pl
from jax.experimental.pallas import tpu as pltpu
from jax.experimental.pallas import tpu_sc as plsc

_CHUNK = 16
_NBUF = 4
_NSLOT = 2


def _embedding_lookup(tokens_flat, table):
    B, = tokens_flat.shape
    V, D = table.shape
    info = plsc.get_sparse_core_info()
    NC, NS = info.num_cores, info.num_subcores
    NW = NC * NS
    b_per_w = B // NW
    n_chunks = b_per_w // _CHUNK
    assert B == NW * b_per_w and b_per_w == n_chunks * _CHUNK
    assert n_chunks % _NBUF == 0 and _NBUF >= _NSLOT

    mesh = plsc.VectorSubcoreMesh(core_axis_name="c", subcore_axis_name="s")

    @functools.partial(
        pl.kernel,
        mesh=mesh,
        out_type=jax.ShapeDtypeStruct((B, D), jnp.float32),
        scratch_types=[
            pltpu.VMEM((b_per_w,), jnp.int32),
        ]
        + [pltpu.VMEM((_CHUNK, D), jnp.float32)] * _NBUF
        + [pltpu.VMEM_SHARED((NS, _NSLOT, _CHUNK, D), jnp.float32)]
        + [pltpu.SemaphoreType.DMA] * (2 * _NBUF + _NSLOT),
    )
    def gather_kernel(idx_hbm, table_hbm, out_hbm, idx_v, *bufs_sems):
        bufs = bufs_sems[:_NBUF]
        shared = bufs_sems[_NBUF]
        gsems = bufs_sems[_NBUF + 1:2 * _NBUF + 1]
        ssems = bufs_sems[2 * _NBUF + 1:3 * _NBUF + 1]
        dsems = bufs_sems[3 * _NBUF + 1:]
        sid = lax.axis_index("s")
        wid = sid * NC + lax.axis_index("c")
        base = wid * b_per_w
        pltpu.sync_copy(idx_hbm.at[pl.ds(base, b_per_w)], idx_v)

        def out_slice(i):
            return out_hbm.at[pl.ds(base + i * _CHUNK, _CHUNK)]

        def slot(s):
            return shared.at[sid, s]

        def start_gather(i, b):
            off = pl.multiple_of(i * _CHUNK, _CHUNK)
            pltpu.async_copy(table_hbm.at[idx_v.at[pl.ds(off, _CHUNK)]],
                             bufs[b], gsems[b])

        for b in range(_NBUF - 1):
            start_gather(b, b)

        def step(i, b, s, first=False, last=False):
            pltpu.make_async_copy(table_hbm.at[idx_v.at[pl.ds(0, _CHUNK)]],
                                  bufs[b], gsems[b]).wait()
            if not last:
                start_gather(i + _NBUF - 1, (b + _NBUF - 1) % _NBUF)
            if not first:
                pltpu.make_async_copy(slot(s), out_slice(0), dsems[s]).wait()
            pltpu.async_copy(bufs[b], slot(s), ssems[b])
            pltpu.make_async_copy(bufs[b], slot(s), ssems[b]).wait()
            pltpu.async_copy(slot(s), out_slice(i), dsems[s])

        for i in range(_NSLOT):
            step(i, i % _NBUF, i % _NSLOT, first=True)

        n_steady = (n_chunks - 2 * _NBUF) // _NBUF

        def body(grp, carry):
            for k in range(_NBUF):
                i = _NSLOT + _NBUF * grp + k
                step(i, (_NSLOT + k) % _NBUF, (_NSLOT + k) % _NSLOT)
            return carry

        lax.fori_loop(0, n_steady, body, 0)

        for i in range(_NSLOT + n_steady * _NBUF, n_chunks):
            step(i, i % _NBUF, i % _NSLOT, last=(i + _NBUF - 1 >= n_chunks))

        for s in range(_NSLOT):
            pltpu.make_async_copy(slot(s), out_slice(0), dsems[s]).wait()

    return gather_kernel(tokens_flat, table)


def kernel(tokens, start_pos, tok_embeddings_weight):
    B, S = tokens.shape
    V, D = tok_embeddings_weight.shape
    out = _embedding_lookup(tokens.reshape(B * S), tok_embeddings_weight)
    return out.reshape(B, S, D)
